# baseline (speedup 1.0000x reference)
import functools

import jax
import jax.numpy as jnp
from jax import lax
from jax.experimental import pallas as pl
from jax.experimental.pallas import tpu as pltpu
from jax.experimental.pallas import tpu_sc as plsc

_VOCAB = 1000000
_EMBED_DIM = 64
_BATCH = 1024
_SEQ = 200

_NUM_WORKERS = 32
_CHUNK = 128
_TOTAL = _BATCH * _SEQ
_CHUNKS_PER_W = _TOTAL // (_NUM_WORKERS * _CHUNK)
_ROWS_PER_W = _CHUNKS_PER_W * _CHUNK


_VB2 = 3906
_VB2_MAIN = 3904
_TAIL_V = 999936


def _iota16():
    return lax.broadcasted_iota(jnp.int32, (16,), 0)


def _tp_block(blk, trans, n_q, n_d=_EMBED_DIM):
    it = _iota16()

    def body(q, _):
        for p in range(2):
            col = jnp.full((16,), 2 * q + p, jnp.int32)
            for d0 in range(0, n_d, 16):
                v = plsc.load_gather(blk, [it + d0, col])
                trans[q, pl.ds(p * n_d + d0, 16)] = v
        return 0

    lax.fori_loop(0, n_q, body, 0)


def _sc_table_body(tt_hbm, out_hbm, blk, trans, blk_t, trans_t, sem):
    nc = 2
    w = lax.axis_index("s") * nc + lax.axis_index("c")
    pltpu.async_copy(
        tt_hbm.at[:, pl.ds(pl.multiple_of(w * 256, 256), 256)], blk, sem
    )

    def body(i, _):
        g = w + 32 * i
        src = tt_hbm.at[:, pl.ds(pl.multiple_of(g * 256, 256), 256)]
        pltpu.make_async_copy(src, blk, sem).wait()
        _tp_block(blk, trans, 128)

        @pl.when(i + 1 < 122)
        def _():
            g2 = w + 32 * (i + 1)
            pltpu.async_copy(
                tt_hbm.at[:, pl.ds(pl.multiple_of(g2 * 256, 256), 256)], blk, sem
            )

        pltpu.sync_copy(
            trans, out_hbm.at[pl.ds(pl.multiple_of(g * 128, 128), 128), :]
        )
        return 0

    lax.fori_loop(0, 122, body, 0)

    @pl.when(w < 2)
    def _():
        g = _VB2_MAIN + w
        pltpu.sync_copy(
            tt_hbm.at[:, pl.ds(pl.multiple_of(g * 256, 256), 256)], blk
        )
        _tp_block(blk, trans, 128)
        pltpu.sync_copy(
            trans, out_hbm.at[pl.ds(pl.multiple_of(g * 128, 128), 128), :]
        )

    @pl.when(w == 2)
    def _():
        pltpu.sync_copy(tt_hbm.at[:, pl.ds(_TAIL_V, 64)], blk_t)
        _tp_block(blk_t, trans_t, 32)
        pltpu.sync_copy(trans_t, out_hbm.at[pl.ds(_TAIL_V // 2, 32), :])


@functools.cache
def _sc_table():
    return pl.kernel(
        _sc_table_body,
        out_type=jax.ShapeDtypeStruct((_VOCAB // 2, 128), jnp.float32),
        mesh=plsc.VectorSubcoreMesh(core_axis_name="c", subcore_axis_name="s"),
        scratch_types=[
            pltpu.VMEM((_EMBED_DIM, 256), jnp.float32),
            pltpu.VMEM((128, 128), jnp.float32),
            pltpu.VMEM((_EMBED_DIM, 64), jnp.float32),
            pltpu.VMEM((32, 128), jnp.float32),
            pltpu.SemaphoreType.DMA,
        ],
        compiler_params=pltpu.CompilerParams(
            use_tc_tiling_on_sc=True, needs_layout_passes=False
        ),
    )


def _sc_gather_body(idx_hbm, table_hbm, out_hbm, idx_v, rows_v, sem):
    nc = 2
    wid = lax.axis_index("s") * nc + lax.axis_index("c")
    row_base = wid * _ROWS_PER_W
    pltpu.sync_copy(idx_hbm.at[pl.ds(row_base, _ROWS_PER_W)], idx_v)

    def body(j, _):
        pltpu.async_copy(
            table_hbm.at[idx_v.at[pl.ds(j * _CHUNK, _CHUNK)]], rows_v, sem
        ).wait()
        pltpu.sync_copy(rows_v, out_hbm.at[pl.ds(row_base + j * _CHUNK, _CHUNK)])
        return 0

    lax.fori_loop(0, _CHUNKS_PER_W, body, 0)


@functools.cache
def _sc_gather():
    return pl.kernel(
        _sc_gather_body,
        out_type=jax.ShapeDtypeStruct((_TOTAL, _EMBED_DIM), jnp.float32),
        mesh=plsc.VectorSubcoreMesh(core_axis_name="c", subcore_axis_name="s"),
        scratch_types=[
            pltpu.VMEM((_ROWS_PER_W,), jnp.int32),
            pltpu.VMEM((_CHUNK, _EMBED_DIM), jnp.float32),
            pltpu.SemaphoreType.DMA,
        ],
        compiler_params=pltpu.CompilerParams(use_tc_tiling_on_sc=False),
    )


_QB = 2


def _tc_mask_body(mask_ref, idx_ref, attn_ref, loss_ref):
    keep = idx_ref[...] != 0
    attn_ref[...] = mask_ref[...] * keep[None].astype(jnp.float32)
    loss_ref[...] = keep


def _tc_mask(mask_t, idx_t):
    return pl.pallas_call(
        _tc_mask_body,
        grid=(_SEQ // _QB,),
        in_specs=[
            pl.BlockSpec((_QB, _SEQ, _BATCH), lambda i: (i, 0, 0)),
            pl.BlockSpec((_SEQ, _BATCH), lambda i: (0, 0)),
        ],
        out_specs=[
            pl.BlockSpec((_QB, _SEQ, _BATCH), lambda i: (i, 0, 0)),
            pl.BlockSpec((_SEQ, _BATCH), lambda i: (0, 0)),
        ],
        out_shape=[
            jax.ShapeDtypeStruct((_SEQ, _SEQ, _BATCH), jnp.float32),
            jax.ShapeDtypeStruct((_SEQ, _BATCH), jnp.bool_),
        ],
    )(mask_t, idx_t)


def kernel(inputs, mask, table):
    idx_flat = inputs.reshape(_TOTAL)
    tl2 = _sc_table()(table.T)
    x_flat = _sc_gather()(idx_flat, tl2.reshape(_VOCAB, _EMBED_DIM))
    mask_t = jnp.transpose(mask.reshape(_BATCH, _SEQ, _SEQ), (1, 2, 0))
    attn_t, loss_t = _tc_mask(mask_t, inputs.T)
    attn = jnp.transpose(attn_t, (2, 0, 1)).reshape(_BATCH, 1, _SEQ, _SEQ)
    return (
        x_flat.reshape(_BATCH, _SEQ, _EMBED_DIM),
        attn,
        loss_t.T,
    )


# speedup vs baseline: 1.1631x; 1.1631x over previous
import functools

import jax
import jax.numpy as jnp
from jax import lax
from jax.experimental import pallas as pl
from jax.experimental.pallas import tpu as pltpu
from jax.experimental.pallas import tpu_sc as plsc

_VOCAB = 1000000
_EMBED_DIM = 64
_BATCH = 1024
_SEQ = 200

_NUM_WORKERS = 32
_CHUNK = 128
_TOTAL = _BATCH * _SEQ
_CHUNKS_PER_W = _TOTAL // (_NUM_WORKERS * _CHUNK)
_ROWS_PER_W = _CHUNKS_PER_W * _CHUNK


_VB2 = 3906
_VB2_MAIN = 3904
_TAIL_V = 999936


def _iota16():
    return lax.broadcasted_iota(jnp.int32, (16,), 0)


def _tp_block(blk, trans, n_c, n_d=_EMBED_DIM):
    it = _iota16()
    r_base = lax.shift_right_logical(it, 1)
    c_par = (it & 1) * n_d
    rows = [r_base + c0 * 8 for c0 in range(n_c // 16)]

    def body(d, _):
        cols = c_par + d
        for c0 in range(n_c // 16):
            v = blk[d, pl.ds(c0 * 16, 16)]
            plsc.store_scatter(trans, [rows[c0], cols], v)
        return 0

    lax.fori_loop(0, n_d, body, 0)


def _sc_table_body(tt_hbm, out_hbm, blk, trans, blk_t, trans_t, sem):
    nc = 2
    w = lax.axis_index("s") * nc + lax.axis_index("c")
    pltpu.async_copy(
        tt_hbm.at[:, pl.ds(pl.multiple_of(w * 256, 256), 256)], blk, sem
    )

    def body(i, _):
        g = w + 32 * i
        src = tt_hbm.at[:, pl.ds(pl.multiple_of(g * 256, 256), 256)]
        pltpu.make_async_copy(src, blk, sem).wait()
        _tp_block(blk, trans, 256)

        @pl.when(i + 1 < 122)
        def _():
            g2 = w + 32 * (i + 1)
            pltpu.async_copy(
                tt_hbm.at[:, pl.ds(pl.multiple_of(g2 * 256, 256), 256)], blk, sem
            )

        pltpu.sync_copy(
            trans, out_hbm.at[pl.ds(pl.multiple_of(g * 128, 128), 128), :]
        )
        return 0

    lax.fori_loop(0, 122, body, 0)

    @pl.when(w < 2)
    def _():
        g = _VB2_MAIN + w
        pltpu.sync_copy(
            tt_hbm.at[:, pl.ds(pl.multiple_of(g * 256, 256), 256)], blk
        )
        _tp_block(blk, trans, 256)
        pltpu.sync_copy(
            trans, out_hbm.at[pl.ds(pl.multiple_of(g * 128, 128), 128), :]
        )

    @pl.when(w == 2)
    def _():
        pltpu.sync_copy(tt_hbm.at[:, pl.ds(_TAIL_V, 64)], blk_t)
        _tp_block(blk_t, trans_t, 64)
        pltpu.sync_copy(trans_t, out_hbm.at[pl.ds(_TAIL_V // 2, 32), :])


@functools.cache
def _sc_table():
    return pl.kernel(
        _sc_table_body,
        out_type=jax.ShapeDtypeStruct((_VOCAB // 2, 128), jnp.float32),
        mesh=plsc.VectorSubcoreMesh(core_axis_name="c", subcore_axis_name="s"),
        scratch_types=[
            pltpu.VMEM((_EMBED_DIM, 256), jnp.float32),
            pltpu.VMEM((128, 128), jnp.float32),
            pltpu.VMEM((_EMBED_DIM, 64), jnp.float32),
            pltpu.VMEM((32, 128), jnp.float32),
            pltpu.SemaphoreType.DMA,
        ],
        compiler_params=pltpu.CompilerParams(
            use_tc_tiling_on_sc=True, needs_layout_passes=False
        ),
    )


def _sc_gather_body(idx_hbm, table_hbm, out_hbm, idx_v, rows_v, sem):
    nc = 2
    wid = lax.axis_index("s") * nc + lax.axis_index("c")
    row_base = wid * _ROWS_PER_W
    pltpu.sync_copy(idx_hbm.at[pl.ds(row_base, _ROWS_PER_W)], idx_v)

    def body(j, _):
        pltpu.async_copy(
            table_hbm.at[idx_v.at[pl.ds(j * _CHUNK, _CHUNK)]], rows_v, sem
        ).wait()
        pltpu.sync_copy(rows_v, out_hbm.at[pl.ds(row_base + j * _CHUNK, _CHUNK)])
        return 0

    lax.fori_loop(0, _CHUNKS_PER_W, body, 0)


@functools.cache
def _sc_gather():
    return pl.kernel(
        _sc_gather_body,
        out_type=jax.ShapeDtypeStruct((_TOTAL, _EMBED_DIM), jnp.float32),
        mesh=plsc.VectorSubcoreMesh(core_axis_name="c", subcore_axis_name="s"),
        scratch_types=[
            pltpu.VMEM((_ROWS_PER_W,), jnp.int32),
            pltpu.VMEM((_CHUNK, _EMBED_DIM), jnp.float32),
            pltpu.SemaphoreType.DMA,
        ],
        compiler_params=pltpu.CompilerParams(use_tc_tiling_on_sc=False),
    )


_QB = 2


def _tc_mask_body(mask_ref, idx_ref, attn_ref, loss_ref):
    keep = idx_ref[...] != 0
    attn_ref[...] = mask_ref[...] * keep[None].astype(jnp.float32)
    loss_ref[...] = keep


def _tc_mask(mask_t, idx_t):
    return pl.pallas_call(
        _tc_mask_body,
        grid=(_SEQ // _QB,),
        in_specs=[
            pl.BlockSpec((_QB, _SEQ, _BATCH), lambda i: (i, 0, 0)),
            pl.BlockSpec((_SEQ, _BATCH), lambda i: (0, 0)),
        ],
        out_specs=[
            pl.BlockSpec((_QB, _SEQ, _BATCH), lambda i: (i, 0, 0)),
            pl.BlockSpec((_SEQ, _BATCH), lambda i: (0, 0)),
        ],
        out_shape=[
            jax.ShapeDtypeStruct((_SEQ, _SEQ, _BATCH), jnp.float32),
            jax.ShapeDtypeStruct((_SEQ, _BATCH), jnp.bool_),
        ],
    )(mask_t, idx_t)


def kernel(inputs, mask, table):
    idx_flat = inputs.reshape(_TOTAL)
    tl2 = _sc_table()(table.T)
    x_flat = _sc_gather()(idx_flat, tl2.reshape(_VOCAB, _EMBED_DIM))
    mask_t = jnp.transpose(mask.reshape(_BATCH, _SEQ, _SEQ), (1, 2, 0))
    attn_t, loss_t = _tc_mask(mask_t, inputs.T)
    attn = jnp.transpose(attn_t, (2, 0, 1)).reshape(_BATCH, 1, _SEQ, _SEQ)
    return (
        x_flat.reshape(_BATCH, _SEQ, _EMBED_DIM),
        attn,
        loss_t.T,
    )


# speedup vs baseline: 1.9450x; 1.6722x over previous
import functools

import jax
import jax.numpy as jnp
from jax import lax
from jax.experimental import pallas as pl
from jax.experimental.pallas import tpu as pltpu
from jax.experimental.pallas import tpu_sc as plsc

_VOCAB = 1000000
_EMBED_DIM = 64
_BATCH = 1024
_SEQ = 200

_NUM_WORKERS = 32
_CHUNK = 128
_TOTAL = _BATCH * _SEQ
_CHUNKS_PER_W = _TOTAL // (_NUM_WORKERS * _CHUNK)
_ROWS_PER_W = _CHUNKS_PER_W * _CHUNK


_VB2 = 3906
_VB2_MAIN = 3904
_TAIL_V = 999936


def _iota16():
    return lax.broadcasted_iota(jnp.int32, (16,), 0)


def _tp_block(blk, trans, n_c, n_d=_EMBED_DIM):
    it = _iota16()
    it_half = lax.shift_right_logical(it, 1)
    par64 = (it & 1) * n_d
    dvecs = [(it + s) & 15 for s in range(16)]
    nd16 = n_d // 16

    def body(m, _):
        d0 = (m % nd16) * 16
        c0 = (m // nd16) * 16
        gcols = c0 + it
        srows = (c0 // 2) + it_half
        for s in range(16):
            dvec = dvecs[s] + d0
            v = plsc.load_gather(blk, [dvec, gcols])
            plsc.store_scatter(trans, [srows, par64 + dvec], v)
        return 0

    lax.fori_loop(0, nd16 * (n_c // 16), body, 0)


def _sc_table_body(tt_hbm, out_hbm, blk, trans, blk_t, trans_t, sem):
    nc = 2
    w = lax.axis_index("s") * nc + lax.axis_index("c")
    pltpu.async_copy(
        tt_hbm.at[:, pl.ds(pl.multiple_of(w * 256, 256), 256)], blk, sem
    )

    def body(i, _):
        g = w + 32 * i
        src = tt_hbm.at[:, pl.ds(pl.multiple_of(g * 256, 256), 256)]
        pltpu.make_async_copy(src, blk, sem).wait()
        _tp_block(blk, trans, 256)

        @pl.when(i + 1 < 122)
        def _():
            g2 = w + 32 * (i + 1)
            pltpu.async_copy(
                tt_hbm.at[:, pl.ds(pl.multiple_of(g2 * 256, 256), 256)], blk, sem
            )

        pltpu.sync_copy(
            trans, out_hbm.at[pl.ds(pl.multiple_of(g * 128, 128), 128), :]
        )
        return 0

    lax.fori_loop(0, 122, body, 0)

    @pl.when(w < 2)
    def _():
        g = _VB2_MAIN + w
        pltpu.sync_copy(
            tt_hbm.at[:, pl.ds(pl.multiple_of(g * 256, 256), 256)], blk
        )
        _tp_block(blk, trans, 256)
        pltpu.sync_copy(
            trans, out_hbm.at[pl.ds(pl.multiple_of(g * 128, 128), 128), :]
        )

    @pl.when(w == 2)
    def _():
        pltpu.sync_copy(tt_hbm.at[:, pl.ds(_TAIL_V, 64)], blk_t)
        _tp_block(blk_t, trans_t, 64)
        pltpu.sync_copy(trans_t, out_hbm.at[pl.ds(_TAIL_V // 2, 32), :])


@functools.cache
def _sc_table():
    return pl.kernel(
        _sc_table_body,
        out_type=jax.ShapeDtypeStruct((_VOCAB // 2, 128), jnp.float32),
        mesh=plsc.VectorSubcoreMesh(core_axis_name="c", subcore_axis_name="s"),
        scratch_types=[
            pltpu.VMEM((_EMBED_DIM, 256), jnp.float32),
            pltpu.VMEM((128, 128), jnp.float32),
            pltpu.VMEM((_EMBED_DIM, 64), jnp.float32),
            pltpu.VMEM((32, 128), jnp.float32),
            pltpu.SemaphoreType.DMA,
        ],
        compiler_params=pltpu.CompilerParams(
            use_tc_tiling_on_sc=True, needs_layout_passes=False
        ),
    )


def _sc_gather_body(idx_hbm, table_hbm, out_hbm, idx_v, rows_v, sem):
    nc = 2
    wid = lax.axis_index("s") * nc + lax.axis_index("c")
    row_base = wid * _ROWS_PER_W
    pltpu.sync_copy(idx_hbm.at[pl.ds(row_base, _ROWS_PER_W)], idx_v)

    def body(j, _):
        pltpu.async_copy(
            table_hbm.at[idx_v.at[pl.ds(j * _CHUNK, _CHUNK)]], rows_v, sem
        ).wait()
        pltpu.sync_copy(rows_v, out_hbm.at[pl.ds(row_base + j * _CHUNK, _CHUNK)])
        return 0

    lax.fori_loop(0, _CHUNKS_PER_W, body, 0)


@functools.cache
def _sc_gather():
    return pl.kernel(
        _sc_gather_body,
        out_type=jax.ShapeDtypeStruct((_TOTAL, _EMBED_DIM), jnp.float32),
        mesh=plsc.VectorSubcoreMesh(core_axis_name="c", subcore_axis_name="s"),
        scratch_types=[
            pltpu.VMEM((_ROWS_PER_W,), jnp.int32),
            pltpu.VMEM((_CHUNK, _EMBED_DIM), jnp.float32),
            pltpu.SemaphoreType.DMA,
        ],
        compiler_params=pltpu.CompilerParams(use_tc_tiling_on_sc=False),
    )


_QB = 2


def _tc_mask_body(mask_ref, idx_ref, attn_ref, loss_ref):
    keep = idx_ref[...] != 0
    attn_ref[...] = mask_ref[...] * keep[None].astype(jnp.float32)
    loss_ref[...] = keep


def _tc_mask(mask_t, idx_t):
    return pl.pallas_call(
        _tc_mask_body,
        grid=(_SEQ // _QB,),
        in_specs=[
            pl.BlockSpec((_QB, _SEQ, _BATCH), lambda i: (i, 0, 0)),
            pl.BlockSpec((_SEQ, _BATCH), lambda i: (0, 0)),
        ],
        out_specs=[
            pl.BlockSpec((_QB, _SEQ, _BATCH), lambda i: (i, 0, 0)),
            pl.BlockSpec((_SEQ, _BATCH), lambda i: (0, 0)),
        ],
        out_shape=[
            jax.ShapeDtypeStruct((_SEQ, _SEQ, _BATCH), jnp.float32),
            jax.ShapeDtypeStruct((_SEQ, _BATCH), jnp.bool_),
        ],
    )(mask_t, idx_t)


def kernel(inputs, mask, table):
    idx_flat = inputs.reshape(_TOTAL)
    tl2 = _sc_table()(table.T)
    x_flat = _sc_gather()(idx_flat, tl2.reshape(_VOCAB, _EMBED_DIM))
    mask_t = jnp.transpose(mask.reshape(_BATCH, _SEQ, _SEQ), (1, 2, 0))
    attn_t, loss_t = _tc_mask(mask_t, inputs.T)
    attn = jnp.transpose(attn_t, (2, 0, 1)).reshape(_BATCH, 1, _SEQ, _SEQ)
    return (
        x_flat.reshape(_BATCH, _SEQ, _EMBED_DIM),
        attn,
        loss_t.T,
    )


# speedup vs baseline: 2.2260x; 1.1445x over previous
import functools

import jax
import jax.numpy as jnp
from jax import lax
from jax.experimental import pallas as pl
from jax.experimental.pallas import tpu as pltpu
from jax.experimental.pallas import tpu_sc as plsc

_VOCAB = 1000000
_EMBED_DIM = 64
_BATCH = 1024
_SEQ = 200

_NUM_WORKERS = 32
_CHUNK = 128
_TOTAL = _BATCH * _SEQ
_CHUNKS_PER_W = _TOTAL // (_NUM_WORKERS * _CHUNK)
_ROWS_PER_W = _CHUNKS_PER_W * _CHUNK


_VB2 = 3906
_VB2_MAIN = 3904
_TAIL_V = 999936


def _iota16():
    return lax.broadcasted_iota(jnp.int32, (16,), 0)


def _tp_block(blk, trans, n_c, n_d=_EMBED_DIM):
    it = _iota16()
    it_half = lax.shift_right_logical(it, 1)
    par64 = (it & 1) * n_d
    dvecs = [(it + s) & 15 for s in range(16)]
    nd16 = n_d // 16

    def body(m, _):
        d0 = (m % nd16) * 16
        c0 = (m // nd16) * 16
        gcols = c0 + it
        srows = (c0 // 2) + it_half
        for s0 in range(0, 16, 8):
            dvs = [dvecs[s0 + u] + d0 for u in range(8)]
            vs = [plsc.load_gather(blk, [dv, gcols]) for dv in dvs]
            for u in range(8):
                plsc.store_scatter(trans, [srows, par64 + dvs[u]], vs[u])
        return 0

    lax.fori_loop(0, nd16 * (n_c // 16), body, 0)


def _sc_table_body(tt_hbm, out_hbm, blk, trans, blk_t, trans_t, sem):
    nc = 2
    w = lax.axis_index("s") * nc + lax.axis_index("c")
    pltpu.async_copy(
        tt_hbm.at[:, pl.ds(pl.multiple_of(w * 256, 256), 256)], blk, sem
    )

    def body(i, _):
        g = w + 32 * i
        src = tt_hbm.at[:, pl.ds(pl.multiple_of(g * 256, 256), 256)]
        pltpu.make_async_copy(src, blk, sem).wait()
        _tp_block(blk, trans, 256)

        @pl.when(i + 1 < 122)
        def _():
            g2 = w + 32 * (i + 1)
            pltpu.async_copy(
                tt_hbm.at[:, pl.ds(pl.multiple_of(g2 * 256, 256), 256)], blk, sem
            )

        pltpu.sync_copy(
            trans, out_hbm.at[pl.ds(pl.multiple_of(g * 128, 128), 128), :]
        )
        return 0

    lax.fori_loop(0, 122, body, 0)

    @pl.when(w < 2)
    def _():
        g = _VB2_MAIN + w
        pltpu.sync_copy(
            tt_hbm.at[:, pl.ds(pl.multiple_of(g * 256, 256), 256)], blk
        )
        _tp_block(blk, trans, 256)
        pltpu.sync_copy(
            trans, out_hbm.at[pl.ds(pl.multiple_of(g * 128, 128), 128), :]
        )

    @pl.when(w == 2)
    def _():
        pltpu.sync_copy(tt_hbm.at[:, pl.ds(_TAIL_V, 64)], blk_t)
        _tp_block(blk_t, trans_t, 64)
        pltpu.sync_copy(trans_t, out_hbm.at[pl.ds(_TAIL_V // 2, 32), :])


@functools.cache
def _sc_table():
    return pl.kernel(
        _sc_table_body,
        out_type=jax.ShapeDtypeStruct((_VOCAB // 2, 128), jnp.float32),
        mesh=plsc.VectorSubcoreMesh(core_axis_name="c", subcore_axis_name="s"),
        scratch_types=[
            pltpu.VMEM((_EMBED_DIM, 256), jnp.float32),
            pltpu.VMEM((128, 128), jnp.float32),
            pltpu.VMEM((_EMBED_DIM, 64), jnp.float32),
            pltpu.VMEM((32, 128), jnp.float32),
            pltpu.SemaphoreType.DMA,
        ],
        compiler_params=pltpu.CompilerParams(
            use_tc_tiling_on_sc=True, needs_layout_passes=False
        ),
    )


def _sc_gather_body(idx_hbm, table_hbm, out_hbm, idx_v, rows_v, sem):
    nc = 2
    wid = lax.axis_index("s") * nc + lax.axis_index("c")
    row_base = wid * _ROWS_PER_W
    pltpu.sync_copy(idx_hbm.at[pl.ds(row_base, _ROWS_PER_W)], idx_v)

    def body(j, _):
        pltpu.async_copy(
            table_hbm.at[idx_v.at[pl.ds(j * _CHUNK, _CHUNK)]], rows_v, sem
        ).wait()
        pltpu.sync_copy(rows_v, out_hbm.at[pl.ds(row_base + j * _CHUNK, _CHUNK)])
        return 0

    lax.fori_loop(0, _CHUNKS_PER_W, body, 0)


@functools.cache
def _sc_gather():
    return pl.kernel(
        _sc_gather_body,
        out_type=jax.ShapeDtypeStruct((_TOTAL, _EMBED_DIM), jnp.float32),
        mesh=plsc.VectorSubcoreMesh(core_axis_name="c", subcore_axis_name="s"),
        scratch_types=[
            pltpu.VMEM((_ROWS_PER_W,), jnp.int32),
            pltpu.VMEM((_CHUNK, _EMBED_DIM), jnp.float32),
            pltpu.SemaphoreType.DMA,
        ],
        compiler_params=pltpu.CompilerParams(use_tc_tiling_on_sc=False),
    )


_QB = 2


def _tc_mask_body(mask_ref, idx_ref, attn_ref, loss_ref):
    keep = idx_ref[...] != 0
    attn_ref[...] = mask_ref[...] * keep[None].astype(jnp.float32)
    loss_ref[...] = keep


def _tc_mask(mask_t, idx_t):
    return pl.pallas_call(
        _tc_mask_body,
        grid=(_SEQ // _QB,),
        in_specs=[
            pl.BlockSpec((_QB, _SEQ, _BATCH), lambda i: (i, 0, 0)),
            pl.BlockSpec((_SEQ, _BATCH), lambda i: (0, 0)),
        ],
        out_specs=[
            pl.BlockSpec((_QB, _SEQ, _BATCH), lambda i: (i, 0, 0)),
            pl.BlockSpec((_SEQ, _BATCH), lambda i: (0, 0)),
        ],
        out_shape=[
            jax.ShapeDtypeStruct((_SEQ, _SEQ, _BATCH), jnp.float32),
            jax.ShapeDtypeStruct((_SEQ, _BATCH), jnp.bool_),
        ],
    )(mask_t, idx_t)


def kernel(inputs, mask, table):
    idx_flat = inputs.reshape(_TOTAL)
    tl2 = _sc_table()(table.T)
    x_flat = _sc_gather()(idx_flat, tl2.reshape(_VOCAB, _EMBED_DIM))
    mask_t = jnp.transpose(mask.reshape(_BATCH, _SEQ, _SEQ), (1, 2, 0))
    attn_t, loss_t = _tc_mask(mask_t, inputs.T)
    attn = jnp.transpose(attn_t, (2, 0, 1)).reshape(_BATCH, 1, _SEQ, _SEQ)
    return (
        x_flat.reshape(_BATCH, _SEQ, _EMBED_DIM),
        attn,
        loss_t.T,
    )


# speedup vs baseline: 3.3410x; 1.5009x over previous
import functools

import jax
import jax.numpy as jnp
from jax import lax
from jax.experimental import pallas as pl
from jax.experimental.pallas import tpu as pltpu
from jax.experimental.pallas import tpu_sc as plsc

_VOCAB = 1000000
_EMBED_DIM = 64
_BATCH = 1024
_SEQ = 200

_NUM_WORKERS = 32
_CHUNK = 128
_TOTAL = _BATCH * _SEQ
_CHUNKS_PER_W = _TOTAL // (_NUM_WORKERS * _CHUNK)
_ROWS_PER_W = _CHUNKS_PER_W * _CHUNK


_VB2 = 3906
_VB2_MAIN = 3904
_TAIL_V = 999936


def _iota16():
    return lax.broadcasted_iota(jnp.int32, (16,), 0)


def _tp_block(blk, trans, n_c, n_d=_EMBED_DIM):
    it = _iota16()
    it_half = lax.shift_right_logical(it, 1)
    par64 = (it & 1) * n_d
    dvecs = [(it + s) & 15 for s in range(16)]
    nd16 = n_d // 16

    def body(m, _):
        d0 = (m & (nd16 - 1)) * 16
        c0 = lax.shift_right_logical(m, nd16.bit_length() - 1) * 16
        gcols = c0 + it
        srows = lax.shift_right_logical(c0, 1) + it_half
        for s0 in range(0, 16, 8):
            dvs = [dvecs[s0 + u] + d0 for u in range(8)]
            vs = [plsc.load_gather(blk, [dv, gcols]) for dv in dvs]
            for u in range(8):
                plsc.store_scatter(trans, [srows, par64 + dvs[u]], vs[u])
        return 0

    lax.fori_loop(0, nd16 * (n_c // 16), body, 0)


def _sc_table_body(tt_hbm, out_hbm, blk0, blk1, trans0, trans1, blk_t, trans_t,
                   sem_in, sem_out):
    nc = 2
    w = lax.axis_index("s") * nc + lax.axis_index("c")
    blks = (blk0, blk1)
    transs = (trans0, trans1)

    def src_at(g):
        return tt_hbm.at[:, pl.ds(pl.multiple_of(g * 256, 256), 256)]

    def dst_at(g):
        return out_hbm.at[pl.ds(pl.multiple_of(g * 128, 128), 128), :]

    pltpu.async_copy(src_at(w), blk0, sem_in)
    pltpu.async_copy(src_at(w + 32), blk1, sem_in)

    def pair(t, _):
        for b in range(2):
            i = 2 * t + b
            g = w + 32 * i
            pltpu.make_async_copy(src_at(g), blks[b], sem_in).wait()

            @pl.when(i >= 2)
            def _():
                pltpu.make_async_copy(
                    transs[b], dst_at(w + 32 * (i - 2)), sem_out
                ).wait()

            _tp_block(blks[b], transs[b], 256)
            pltpu.async_copy(transs[b], dst_at(g), sem_out)

            @pl.when(i + 2 < 122)
            def _():
                pltpu.async_copy(src_at(w + 32 * (i + 2)), blks[b], sem_in)
        return 0

    lax.fori_loop(0, 61, pair, 0)
    pltpu.make_async_copy(trans0, dst_at(w + 32 * 120), sem_out).wait()
    pltpu.make_async_copy(trans1, dst_at(w + 32 * 121), sem_out).wait()

    @pl.when(w < 2)
    def _():
        g = _VB2_MAIN + w
        pltpu.sync_copy(src_at(g), blk0)
        _tp_block(blk0, trans0, 256)
        pltpu.sync_copy(trans0, dst_at(g))

    @pl.when(w == 2)
    def _():
        pltpu.sync_copy(tt_hbm.at[:, pl.ds(_TAIL_V, 64)], blk_t)
        _tp_block(blk_t, trans_t, 64)
        pltpu.sync_copy(trans_t, out_hbm.at[pl.ds(_TAIL_V // 2, 32), :])


@functools.cache
def _sc_table():
    return pl.kernel(
        _sc_table_body,
        out_type=jax.ShapeDtypeStruct((_VOCAB // 2, 128), jnp.float32),
        mesh=plsc.VectorSubcoreMesh(core_axis_name="c", subcore_axis_name="s"),
        scratch_types=[
            pltpu.VMEM((_EMBED_DIM, 256), jnp.float32),
            pltpu.VMEM((_EMBED_DIM, 256), jnp.float32),
            pltpu.VMEM((128, 128), jnp.float32),
            pltpu.VMEM((128, 128), jnp.float32),
            pltpu.VMEM((_EMBED_DIM, 64), jnp.float32),
            pltpu.VMEM((32, 128), jnp.float32),
            pltpu.SemaphoreType.DMA,
            pltpu.SemaphoreType.DMA,
        ],
        compiler_params=pltpu.CompilerParams(
            use_tc_tiling_on_sc=True, needs_layout_passes=False
        ),
    )


def _sc_gather_body(idx_hbm, table_hbm, out_hbm, idx_v, rows_v, sem):
    nc = 2
    wid = lax.axis_index("s") * nc + lax.axis_index("c")
    row_base = wid * _ROWS_PER_W
    pltpu.sync_copy(idx_hbm.at[pl.ds(row_base, _ROWS_PER_W)], idx_v)

    def body(j, _):
        pltpu.async_copy(
            table_hbm.at[idx_v.at[pl.ds(j * _CHUNK, _CHUNK)]], rows_v, sem
        ).wait()
        pltpu.sync_copy(rows_v, out_hbm.at[pl.ds(row_base + j * _CHUNK, _CHUNK)])
        return 0

    lax.fori_loop(0, _CHUNKS_PER_W, body, 0)


@functools.cache
def _sc_gather():
    return pl.kernel(
        _sc_gather_body,
        out_type=jax.ShapeDtypeStruct((_TOTAL, _EMBED_DIM), jnp.float32),
        mesh=plsc.VectorSubcoreMesh(core_axis_name="c", subcore_axis_name="s"),
        scratch_types=[
            pltpu.VMEM((_ROWS_PER_W,), jnp.int32),
            pltpu.VMEM((_CHUNK, _EMBED_DIM), jnp.float32),
            pltpu.SemaphoreType.DMA,
        ],
        compiler_params=pltpu.CompilerParams(use_tc_tiling_on_sc=False),
    )


_QB = 2


def _tc_mask_body(mask_ref, idx_ref, attn_ref, loss_ref):
    keep = idx_ref[...] != 0
    attn_ref[...] = mask_ref[...] * keep[None].astype(jnp.float32)
    loss_ref[...] = keep


def _tc_mask(mask_t, idx_t):
    return pl.pallas_call(
        _tc_mask_body,
        grid=(_SEQ // _QB,),
        in_specs=[
            pl.BlockSpec((_QB, _SEQ, _BATCH), lambda i: (i, 0, 0)),
            pl.BlockSpec((_SEQ, _BATCH), lambda i: (0, 0)),
        ],
        out_specs=[
            pl.BlockSpec((_QB, _SEQ, _BATCH), lambda i: (i, 0, 0)),
            pl.BlockSpec((_SEQ, _BATCH), lambda i: (0, 0)),
        ],
        out_shape=[
            jax.ShapeDtypeStruct((_SEQ, _SEQ, _BATCH), jnp.float32),
            jax.ShapeDtypeStruct((_SEQ, _BATCH), jnp.bool_),
        ],
    )(mask_t, idx_t)


def kernel(inputs, mask, table):
    idx_flat = inputs.reshape(_TOTAL)
    tl2 = _sc_table()(table.T)
    x_flat = _sc_gather()(idx_flat, tl2.reshape(_VOCAB, _EMBED_DIM))
    mask_t = jnp.transpose(mask.reshape(_BATCH, _SEQ, _SEQ), (1, 2, 0))
    attn_t, loss_t = _tc_mask(mask_t, inputs.T)
    attn = jnp.transpose(attn_t, (2, 0, 1)).reshape(_BATCH, 1, _SEQ, _SEQ)
    return (
        x_flat.reshape(_BATCH, _SEQ, _EMBED_DIM),
        attn,
        loss_t.T,
    )


# speedup vs baseline: 4.1359x; 1.2379x over previous
import functools

import jax
import jax.numpy as jnp
from jax import lax
from jax.experimental import pallas as pl
from jax.experimental.pallas import tpu as pltpu
from jax.experimental.pallas import tpu_sc as plsc

_VOCAB = 1000000
_EMBED_DIM = 64
_BATCH = 1024
_SEQ = 200

_NUM_WORKERS = 32
_CHUNK = 128
_TOTAL = _BATCH * _SEQ
_CHUNKS_PER_W = _TOTAL // (_NUM_WORKERS * _CHUNK)
_ROWS_PER_W = _CHUNKS_PER_W * _CHUNK


_VB2 = 3906
_VB2_MAIN = 3904
_TAIL_V = 999936


def _iota16():
    return lax.broadcasted_iota(jnp.int32, (16,), 0)


def _tp_block(blk, trans, n_c, n_d=_EMBED_DIM):
    it = _iota16()
    it_half = lax.shift_right_logical(it, 1)
    par64 = (it & 1) * n_d
    dvecs = [(it + s) & 15 for s in range(16)]
    nd16 = n_d // 16

    def body(m, _):
        d0 = (m & (nd16 - 1)) * 16
        c0 = lax.shift_right_logical(m, nd16.bit_length() - 1) * 16
        gcols = c0 + it
        srows = lax.shift_right_logical(c0, 1) + it_half
        for s0 in range(0, 16, 8):
            dvs = [dvecs[s0 + u] + d0 for u in range(8)]
            vs = [plsc.load_gather(blk, [dv, gcols]) for dv in dvs]
            for u in range(8):
                plsc.store_scatter(trans, [srows, par64 + dvs[u]], vs[u])
        return 0

    lax.fori_loop(0, nd16 * (n_c // 16), body, 0)


def _sc_table_body(tt_hbm, out_hbm, blk0, blk1, trans0, trans1, blk_t, trans_t,
                   sem_in, sem_out):
    nc = 2
    w = lax.axis_index("s") * nc + lax.axis_index("c")
    blks = (blk0, blk1)
    transs = (trans0, trans1)

    def src_at(g):
        return tt_hbm.at[:, pl.ds(pl.multiple_of(g * 256, 256), 256)]

    def dst_at(g):
        return out_hbm.at[pl.ds(pl.multiple_of(g * 128, 128), 128), :]

    pltpu.async_copy(src_at(w), blk0, sem_in)
    pltpu.async_copy(src_at(w + 32), blk1, sem_in)

    def pair(t, _):
        for b in range(2):
            i = 2 * t + b
            g = w + 32 * i
            pltpu.make_async_copy(src_at(g), blks[b], sem_in).wait()

            @pl.when(i >= 2)
            def _():
                pltpu.make_async_copy(
                    transs[b], dst_at(w + 32 * (i - 2)), sem_out
                ).wait()

            _tp_block(blks[b], transs[b], 256)
            pltpu.async_copy(transs[b], dst_at(g), sem_out)

            @pl.when(i + 2 < 122)
            def _():
                pltpu.async_copy(src_at(w + 32 * (i + 2)), blks[b], sem_in)
        return 0

    lax.fori_loop(0, 61, pair, 0)
    pltpu.make_async_copy(trans0, dst_at(w + 32 * 120), sem_out).wait()
    pltpu.make_async_copy(trans1, dst_at(w + 32 * 121), sem_out).wait()

    @pl.when(w < 2)
    def _():
        g = _VB2_MAIN + w
        pltpu.sync_copy(src_at(g), blk0)
        _tp_block(blk0, trans0, 256)
        pltpu.sync_copy(trans0, dst_at(g))

    @pl.when(w == 2)
    def _():
        pltpu.sync_copy(tt_hbm.at[:, pl.ds(_TAIL_V, 64)], blk_t)
        _tp_block(blk_t, trans_t, 64)
        pltpu.sync_copy(trans_t, out_hbm.at[pl.ds(_TAIL_V // 2, 32), :])


@functools.cache
def _sc_table():
    return pl.kernel(
        _sc_table_body,
        out_type=jax.ShapeDtypeStruct((_VOCAB // 2, 128), jnp.float32),
        mesh=plsc.VectorSubcoreMesh(core_axis_name="c", subcore_axis_name="s"),
        scratch_types=[
            pltpu.VMEM((_EMBED_DIM, 256), jnp.float32),
            pltpu.VMEM((_EMBED_DIM, 256), jnp.float32),
            pltpu.VMEM((128, 128), jnp.float32),
            pltpu.VMEM((128, 128), jnp.float32),
            pltpu.VMEM((_EMBED_DIM, 64), jnp.float32),
            pltpu.VMEM((32, 128), jnp.float32),
            pltpu.SemaphoreType.DMA,
            pltpu.SemaphoreType.DMA,
        ],
        compiler_params=pltpu.CompilerParams(
            use_tc_tiling_on_sc=True, needs_layout_passes=False
        ),
    )


def _sc_gather_body(idx_hbm, table_hbm, out_hbm, idx_v, rows_v, sem):
    nc = 2
    wid = lax.axis_index("s") * nc + lax.axis_index("c")
    row_base = wid * _ROWS_PER_W
    pltpu.sync_copy(idx_hbm.at[pl.ds(row_base, _ROWS_PER_W)], idx_v)

    def body(j, _):
        pltpu.async_copy(
            table_hbm.at[idx_v.at[pl.ds(j * _CHUNK, _CHUNK)]], rows_v, sem
        ).wait()
        pltpu.sync_copy(rows_v, out_hbm.at[pl.ds(row_base + j * _CHUNK, _CHUNK)])
        return 0

    lax.fori_loop(0, _CHUNKS_PER_W, body, 0)


@functools.cache
def _sc_gather():
    return pl.kernel(
        _sc_gather_body,
        out_type=jax.ShapeDtypeStruct((_TOTAL, _EMBED_DIM), jnp.float32),
        mesh=plsc.VectorSubcoreMesh(core_axis_name="c", subcore_axis_name="s"),
        scratch_types=[
            pltpu.VMEM((_ROWS_PER_W,), jnp.int32),
            pltpu.VMEM((_CHUNK, _EMBED_DIM), jnp.float32),
            pltpu.SemaphoreType.DMA,
        ],
        compiler_params=pltpu.CompilerParams(use_tc_tiling_on_sc=False),
    )


def _sc_gatherx_body(idx_hbm, table_hbm, out_hbm, idxb, lines, rows_v, xu, sem):
    nc = 2
    w = lax.axis_index("s") * nc + lax.axis_index("c")
    it = _iota16()
    dvecs = [(it + s) & 15 for s in range(16)]

    def unit(t, _):
        u = w + 32 * t

        @pl.when(u < 200)
        def _():
            so = u // 8
            j = u - so * 8
            pltpu.sync_copy(
                idx_hbm.at[
                    pl.ds(pl.multiple_of(so * 8, 8), 8),
                    pl.ds(pl.multiple_of(j * 128, 128), 128),
                ],
                idxb,
            )

            def row(r, _):
                for g in range(8):
                    v = idxb[r, pl.ds(g * 16, 16)]
                    lines[pl.ds(g * 16, 16)] = lax.shift_right_logical(v, 1)
                pltpu.async_copy(table_hbm.at[lines], rows_v, sem).wait()

                def sub(m, _):
                    d0 = (m & 3) * 16
                    l0 = lax.shift_right_logical(m, 2) * 16
                    parv = (idxb[r, pl.ds(l0, 16)] & 1) * 64
                    grows = l0 + it
                    for s0 in range(0, 16, 8):
                        dvs = [dvecs[s0 + q] + d0 for q in range(8)]
                        vs = [
                            plsc.load_gather(rows_v, [grows, parv + dv])
                            for dv in dvs
                        ]
                        for q in range(8):
                            plsc.store_scatter(xu, [dvs[q], grows], vs[q])
                    return 0

                lax.fori_loop(0, 32, sub, 0)
                pltpu.sync_copy(
                    xu,
                    out_hbm.at[
                        so * 8 + r, :, pl.ds(pl.multiple_of(j * 128, 128), 128)
                    ],
                )
                return 0

            lax.fori_loop(0, 8, row, 0)

        return 0

    lax.fori_loop(0, 7, unit, 0)


@functools.cache
def _sc_gatherx():
    return pl.kernel(
        _sc_gatherx_body,
        out_type=jax.ShapeDtypeStruct((_SEQ, _EMBED_DIM, _BATCH), jnp.float32),
        mesh=plsc.VectorSubcoreMesh(core_axis_name="c", subcore_axis_name="s"),
        scratch_types=[
            pltpu.VMEM((8, 128), jnp.int32),
            pltpu.VMEM((128,), jnp.int32),
            pltpu.VMEM((128, 128), jnp.float32),
            pltpu.VMEM((_EMBED_DIM, 128), jnp.float32),
            pltpu.SemaphoreType.DMA,
        ],
        compiler_params=pltpu.CompilerParams(
            use_tc_tiling_on_sc=True, needs_layout_passes=False
        ),
    )


_QB = 2


def _tc_mask_body(mask_ref, idx_ref, attn_ref, loss_ref):
    keep = idx_ref[...] != 0
    attn_ref[...] = mask_ref[...] * keep[None].astype(jnp.float32)
    loss_ref[...] = keep


def _tc_mask(mask_t, idx_t):
    return pl.pallas_call(
        _tc_mask_body,
        grid=(_SEQ // _QB,),
        in_specs=[
            pl.BlockSpec((_QB, _SEQ, _BATCH), lambda i: (i, 0, 0)),
            pl.BlockSpec((_SEQ, _BATCH), lambda i: (0, 0)),
        ],
        out_specs=[
            pl.BlockSpec((_QB, _SEQ, _BATCH), lambda i: (i, 0, 0)),
            pl.BlockSpec((_SEQ, _BATCH), lambda i: (0, 0)),
        ],
        out_shape=[
            jax.ShapeDtypeStruct((_SEQ, _SEQ, _BATCH), jnp.float32),
            jax.ShapeDtypeStruct((_SEQ, _BATCH), jnp.bool_),
        ],
    )(mask_t, idx_t)


def kernel(inputs, mask, table):
    mask_t = jnp.transpose(mask.reshape(_BATCH, _SEQ, _SEQ), (1, 2, 0))
    attn_t, loss_t = _tc_mask(mask_t, inputs.T)
    tl2 = _sc_table()(table.T)
    xt3 = _sc_gatherx()(inputs.T, tl2)
    attn = jnp.transpose(attn_t, (2, 0, 1)).reshape(_BATCH, 1, _SEQ, _SEQ)
    return (
        jnp.transpose(xt3, (2, 0, 1)),
        attn,
        loss_t.T,
    )


# speedup vs baseline: 4.3564x; 1.0533x over previous
import functools

import jax
import jax.numpy as jnp
from jax import lax
from jax.experimental import pallas as pl
from jax.experimental.pallas import tpu as pltpu
from jax.experimental.pallas import tpu_sc as plsc

_VOCAB = 1000000
_EMBED_DIM = 64
_BATCH = 1024
_SEQ = 200

_NUM_WORKERS = 32
_CHUNK = 128
_TOTAL = _BATCH * _SEQ
_CHUNKS_PER_W = _TOTAL // (_NUM_WORKERS * _CHUNK)
_ROWS_PER_W = _CHUNKS_PER_W * _CHUNK


_VB2 = 3906
_VB2_MAIN = 3904
_TAIL_V = 999936


def _iota16():
    return lax.broadcasted_iota(jnp.int32, (16,), 0)


def _tp_block(blk, trans, n_c, n_d=_EMBED_DIM):
    it = _iota16()
    it_half = lax.shift_right_logical(it, 1)
    par64 = (it & 1) * n_d
    dvecs = [(it + s) & 15 for s in range(16)]
    nd16 = n_d // 16

    def body(m, _):
        d0 = (m & (nd16 - 1)) * 16
        c0 = lax.shift_right_logical(m, nd16.bit_length() - 1) * 16
        gcols = c0 + it
        srows = lax.shift_right_logical(c0, 1) + it_half
        for s0 in range(0, 16, 8):
            dvs = [dvecs[s0 + u] + d0 for u in range(8)]
            vs = [plsc.load_gather(blk, [dv, gcols]) for dv in dvs]
            for u in range(8):
                plsc.store_scatter(trans, [srows, par64 + dvs[u]], vs[u])
        return 0

    lax.fori_loop(0, nd16 * (n_c // 16), body, 0)


def _sc_table_body(tt_hbm, out_hbm, blk0, blk1, trans0, trans1, blk_t, trans_t,
                   sem_in, sem_out):
    nc = 2
    w = lax.axis_index("s") * nc + lax.axis_index("c")
    blks = (blk0, blk1)
    transs = (trans0, trans1)

    def src_at(g):
        return tt_hbm.at[:, pl.ds(pl.multiple_of(g * 256, 256), 256)]

    def dst_at(g):
        return out_hbm.at[pl.ds(pl.multiple_of(g * 128, 128), 128), :]

    pltpu.async_copy(src_at(w), blk0, sem_in)
    pltpu.async_copy(src_at(w + 32), blk1, sem_in)

    def pair(t, _):
        for b in range(2):
            i = 2 * t + b
            g = w + 32 * i
            pltpu.make_async_copy(src_at(g), blks[b], sem_in).wait()

            @pl.when(i >= 2)
            def _():
                pltpu.make_async_copy(
                    transs[b], dst_at(w + 32 * (i - 2)), sem_out
                ).wait()

            _tp_block(blks[b], transs[b], 256)
            pltpu.async_copy(transs[b], dst_at(g), sem_out)

            @pl.when(i + 2 < 122)
            def _():
                pltpu.async_copy(src_at(w + 32 * (i + 2)), blks[b], sem_in)
        return 0

    lax.fori_loop(0, 61, pair, 0)
    pltpu.make_async_copy(trans0, dst_at(w + 32 * 120), sem_out).wait()
    pltpu.make_async_copy(trans1, dst_at(w + 32 * 121), sem_out).wait()

    @pl.when(w < 2)
    def _():
        g = _VB2_MAIN + w
        pltpu.sync_copy(src_at(g), blk0)
        _tp_block(blk0, trans0, 256)
        pltpu.sync_copy(trans0, dst_at(g))

    @pl.when(w == 2)
    def _():
        pltpu.sync_copy(tt_hbm.at[:, pl.ds(_TAIL_V, 64)], blk_t)
        _tp_block(blk_t, trans_t, 64)
        pltpu.sync_copy(trans_t, out_hbm.at[pl.ds(_TAIL_V // 2, 32), :])


@functools.cache
def _sc_table():
    return pl.kernel(
        _sc_table_body,
        out_type=jax.ShapeDtypeStruct((_VOCAB // 2, 128), jnp.float32),
        mesh=plsc.VectorSubcoreMesh(core_axis_name="c", subcore_axis_name="s"),
        scratch_types=[
            pltpu.VMEM((_EMBED_DIM, 256), jnp.float32),
            pltpu.VMEM((_EMBED_DIM, 256), jnp.float32),
            pltpu.VMEM((128, 128), jnp.float32),
            pltpu.VMEM((128, 128), jnp.float32),
            pltpu.VMEM((_EMBED_DIM, 64), jnp.float32),
            pltpu.VMEM((32, 128), jnp.float32),
            pltpu.SemaphoreType.DMA,
            pltpu.SemaphoreType.DMA,
        ],
        compiler_params=pltpu.CompilerParams(
            use_tc_tiling_on_sc=True, needs_layout_passes=False
        ),
    )


def _sc_gather_body(idx_hbm, table_hbm, out_hbm, idx_v, rows_v, sem):
    nc = 2
    wid = lax.axis_index("s") * nc + lax.axis_index("c")
    row_base = wid * _ROWS_PER_W
    pltpu.sync_copy(idx_hbm.at[pl.ds(row_base, _ROWS_PER_W)], idx_v)

    def body(j, _):
        pltpu.async_copy(
            table_hbm.at[idx_v.at[pl.ds(j * _CHUNK, _CHUNK)]], rows_v, sem
        ).wait()
        pltpu.sync_copy(rows_v, out_hbm.at[pl.ds(row_base + j * _CHUNK, _CHUNK)])
        return 0

    lax.fori_loop(0, _CHUNKS_PER_W, body, 0)


@functools.cache
def _sc_gather():
    return pl.kernel(
        _sc_gather_body,
        out_type=jax.ShapeDtypeStruct((_TOTAL, _EMBED_DIM), jnp.float32),
        mesh=plsc.VectorSubcoreMesh(core_axis_name="c", subcore_axis_name="s"),
        scratch_types=[
            pltpu.VMEM((_ROWS_PER_W,), jnp.int32),
            pltpu.VMEM((_CHUNK, _EMBED_DIM), jnp.float32),
            pltpu.SemaphoreType.DMA,
        ],
        compiler_params=pltpu.CompilerParams(use_tc_tiling_on_sc=False),
    )


def _sc_gatherx_body(idx_hbm, table_hbm, out_hbm, idxb, lines, lines2, rows_v,
                     rows_v2, xu, sem):
    nc = 2
    w = lax.axis_index("s") * nc + lax.axis_index("c")
    it = _iota16()
    dvecs = [(it + s) & 15 for s in range(16)]

    def unit(t, _):
        u = w + 32 * t

        @pl.when(u < 200)
        def _():
            so = u // 8
            j = u - so * 8
            pltpu.sync_copy(
                idx_hbm.at[
                    pl.ds(pl.multiple_of(so * 8, 8), 8),
                    pl.ds(pl.multiple_of(j * 128, 128), 128),
                ],
                idxb,
            )

            def fill_lines(r, lref):
                for g in range(8):
                    v = idxb[r, pl.ds(g * 16, 16)]
                    lref[pl.ds(g * 16, 16)] = lax.shift_right_logical(v, 1)

            def transpose_row(r, rref):
                def sub(m, _):
                    d0 = (m & 3) * 16
                    l0 = lax.shift_right_logical(m, 2) * 16
                    parv = (idxb[r, pl.ds(l0, 16)] & 1) * 64
                    grows = l0 + it
                    for s0 in range(0, 16, 8):
                        dvs = [dvecs[s0 + q] + d0 for q in range(8)]
                        vs = [
                            plsc.load_gather(rref, [grows, parv + dv])
                            for dv in dvs
                        ]
                        for q in range(8):
                            plsc.store_scatter(xu, [dvs[q], grows], vs[q])
                    return 0

                lax.fori_loop(0, 32, sub, 0)

            liness = (lines, lines2)
            rowss = (rows_v, rows_v2)
            fill_lines(0, lines)
            pltpu.async_copy(table_hbm.at[lines], rows_v, sem)
            fill_lines(1, lines2)
            pltpu.async_copy(table_hbm.at[lines2], rows_v2, sem)

            def pair(t, _):
                for b in range(2):
                    r = 2 * t + b
                    pltpu.make_async_copy(
                        table_hbm.at[liness[b]], rowss[b], sem
                    ).wait()
                    transpose_row(r, rowss[b])

                    @pl.when(t < 3)
                    def _():
                        fill_lines(r + 2, liness[b])
                        pltpu.async_copy(
                            table_hbm.at[liness[b]], rowss[b], sem
                        )

                    pltpu.sync_copy(
                        xu,
                        out_hbm.at[
                            so * 8 + r,
                            :,
                            pl.ds(pl.multiple_of(j * 128, 128), 128),
                        ],
                    )
                return 0

            lax.fori_loop(0, 4, pair, 0)

        return 0

    lax.fori_loop(0, 7, unit, 0)


@functools.cache
def _sc_gatherx():
    return pl.kernel(
        _sc_gatherx_body,
        out_type=jax.ShapeDtypeStruct((_SEQ, _EMBED_DIM, _BATCH), jnp.float32),
        mesh=plsc.VectorSubcoreMesh(core_axis_name="c", subcore_axis_name="s"),
        scratch_types=[
            pltpu.VMEM((8, 128), jnp.int32),
            pltpu.VMEM((128,), jnp.int32),
            pltpu.VMEM((128,), jnp.int32),
            pltpu.VMEM((128, 128), jnp.float32),
            pltpu.VMEM((128, 128), jnp.float32),
            pltpu.VMEM((_EMBED_DIM, 128), jnp.float32),
            pltpu.SemaphoreType.DMA,
        ],
        compiler_params=pltpu.CompilerParams(
            use_tc_tiling_on_sc=True, needs_layout_passes=False
        ),
    )


_QB = 2


def _tc_mask_body(mask_ref, idx_ref, attn_ref, loss_ref):
    keep = idx_ref[...] != 0
    attn_ref[...] = mask_ref[...] * keep[None].astype(jnp.float32)
    loss_ref[...] = keep


def _tc_mask(mask_t, idx_t):
    return pl.pallas_call(
        _tc_mask_body,
        grid=(_SEQ // _QB,),
        in_specs=[
            pl.BlockSpec((_QB, _SEQ, _BATCH), lambda i: (i, 0, 0)),
            pl.BlockSpec((_SEQ, _BATCH), lambda i: (0, 0)),
        ],
        out_specs=[
            pl.BlockSpec((_QB, _SEQ, _BATCH), lambda i: (i, 0, 0)),
            pl.BlockSpec((_SEQ, _BATCH), lambda i: (0, 0)),
        ],
        out_shape=[
            jax.ShapeDtypeStruct((_SEQ, _SEQ, _BATCH), jnp.float32),
            jax.ShapeDtypeStruct((_SEQ, _BATCH), jnp.bool_),
        ],
        cost_estimate=pl.CostEstimate(
            flops=2 * _SEQ * _SEQ * _BATCH,
            bytes_accessed=2 * _SEQ * _SEQ * _BATCH * 4,
            transcendentals=0,
        ),
    )(mask_t, idx_t)


def kernel(inputs, mask, table):
    mask_t = jnp.transpose(mask.reshape(_BATCH, _SEQ, _SEQ), (1, 2, 0))
    attn_t, loss_t = _tc_mask(mask_t, inputs.T)
    tl2 = _sc_table()(table.T)
    xt3 = _sc_gatherx()(inputs.T, tl2)
    attn = jnp.transpose(attn_t, (2, 0, 1)).reshape(_BATCH, 1, _SEQ, _SEQ)
    return (
        jnp.transpose(xt3, (2, 0, 1)),
        attn,
        loss_t.T,
    )


# speedup vs baseline: 4.3625x; 1.0014x over previous
import functools

import jax
import jax.numpy as jnp
from jax import lax
from jax.experimental import pallas as pl
from jax.experimental.pallas import tpu as pltpu
from jax.experimental.pallas import tpu_sc as plsc

_VOCAB = 1000000
_EMBED_DIM = 64
_BATCH = 1024
_SEQ = 200

_NUM_WORKERS = 32
_CHUNK = 128
_TOTAL = _BATCH * _SEQ
_CHUNKS_PER_W = _TOTAL // (_NUM_WORKERS * _CHUNK)
_ROWS_PER_W = _CHUNKS_PER_W * _CHUNK


_VB2 = 3906
_VB2_MAIN = 3904
_TAIL_V = 999936


def _iota16():
    return lax.broadcasted_iota(jnp.int32, (16,), 0)


def _tp_block(blk, trans, n_c, n_d=_EMBED_DIM):
    it = _iota16()
    it_half = lax.shift_right_logical(it, 1)
    par64 = (it & 1) * n_d
    dvecs = [(it + s) & 15 for s in range(16)]
    nd16 = n_d // 16

    def body(m, _):
        d0 = (m & (nd16 - 1)) * 16
        c0 = lax.shift_right_logical(m, nd16.bit_length() - 1) * 16
        gcols = c0 + it
        srows = lax.shift_right_logical(c0, 1) + it_half
        for s0 in range(0, 16, 8):
            dvs = [dvecs[s0 + u] + d0 for u in range(8)]
            vs = [plsc.load_gather(blk, [dv, gcols]) for dv in dvs]
            for u in range(8):
                plsc.store_scatter(trans, [srows, par64 + dvs[u]], vs[u])
        return 0

    lax.fori_loop(0, nd16 * (n_c // 16), body, 0)


def _sc_table_body(tt_hbm, out_hbm, blk0, blk1, trans0, trans1, blk_t, trans_t,
                   sem_in, sem_out):
    nc = 2
    w = lax.axis_index("s") * nc + lax.axis_index("c")
    blks = (blk0, blk1)
    transs = (trans0, trans1)

    def src_at(g):
        return tt_hbm.at[:, pl.ds(pl.multiple_of(g * 256, 256), 256)]

    def dst_at(g):
        return out_hbm.at[pl.ds(pl.multiple_of(g * 128, 128), 128), :]

    pltpu.async_copy(src_at(w), blk0, sem_in)
    pltpu.async_copy(src_at(w + 32), blk1, sem_in)

    def pair(t, _):
        for b in range(2):
            i = 2 * t + b
            g = w + 32 * i
            pltpu.make_async_copy(src_at(g), blks[b], sem_in).wait()

            @pl.when(i >= 2)
            def _():
                pltpu.make_async_copy(
                    transs[b], dst_at(w + 32 * (i - 2)), sem_out
                ).wait()

            _tp_block(blks[b], transs[b], 256)
            pltpu.async_copy(transs[b], dst_at(g), sem_out)

            @pl.when(i + 2 < 122)
            def _():
                pltpu.async_copy(src_at(w + 32 * (i + 2)), blks[b], sem_in)
        return 0

    lax.fori_loop(0, 61, pair, 0)
    pltpu.make_async_copy(trans0, dst_at(w + 32 * 120), sem_out).wait()
    pltpu.make_async_copy(trans1, dst_at(w + 32 * 121), sem_out).wait()

    @pl.when(w < 2)
    def _():
        g = _VB2_MAIN + w
        pltpu.sync_copy(src_at(g), blk0)
        _tp_block(blk0, trans0, 256)
        pltpu.sync_copy(trans0, dst_at(g))

    @pl.when(w == 2)
    def _():
        pltpu.sync_copy(tt_hbm.at[:, pl.ds(_TAIL_V, 64)], blk_t)
        _tp_block(blk_t, trans_t, 64)
        pltpu.sync_copy(trans_t, out_hbm.at[pl.ds(_TAIL_V // 2, 32), :])


@functools.cache
def _sc_table():
    return pl.kernel(
        _sc_table_body,
        out_type=jax.ShapeDtypeStruct((_VOCAB // 2, 128), jnp.float32),
        mesh=plsc.VectorSubcoreMesh(core_axis_name="c", subcore_axis_name="s"),
        scratch_types=[
            pltpu.VMEM((_EMBED_DIM, 256), jnp.float32),
            pltpu.VMEM((_EMBED_DIM, 256), jnp.float32),
            pltpu.VMEM((128, 128), jnp.float32),
            pltpu.VMEM((128, 128), jnp.float32),
            pltpu.VMEM((_EMBED_DIM, 64), jnp.float32),
            pltpu.VMEM((32, 128), jnp.float32),
            pltpu.SemaphoreType.DMA,
            pltpu.SemaphoreType.DMA,
        ],
        compiler_params=pltpu.CompilerParams(
            use_tc_tiling_on_sc=True, needs_layout_passes=False
        ),
    )


def _sc_gatherx_body(idx_hbm, table_hbm, out_hbm, idxb, lines, lines2, rows_v,
                     rows_v2, xu, sem):
    nc = 2
    w = lax.axis_index("s") * nc + lax.axis_index("c")
    it = _iota16()
    dvecs = [(it + s) & 15 for s in range(16)]

    def unit(t, _):
        u = w + 32 * t

        @pl.when(u < 200)
        def _():
            so = u // 8
            j = u - so * 8
            pltpu.sync_copy(
                idx_hbm.at[
                    pl.ds(pl.multiple_of(so * 8, 8), 8),
                    pl.ds(pl.multiple_of(j * 128, 128), 128),
                ],
                idxb,
            )

            def fill_lines(r, lref):
                for g in range(8):
                    v = idxb[r, pl.ds(g * 16, 16)]
                    lref[pl.ds(g * 16, 16)] = lax.shift_right_logical(v, 1)

            def transpose_row(r, rref):
                def sub(m, _):
                    d0 = (m & 3) * 16
                    l0 = lax.shift_right_logical(m, 2) * 16
                    parv = (idxb[r, pl.ds(l0, 16)] & 1) * 64
                    grows = l0 + it
                    for s0 in range(0, 16, 8):
                        dvs = [dvecs[s0 + q] + d0 for q in range(8)]
                        vs = [
                            plsc.load_gather(rref, [grows, parv + dv])
                            for dv in dvs
                        ]
                        for q in range(8):
                            plsc.store_scatter(xu, [dvs[q], grows], vs[q])
                    return 0

                lax.fori_loop(0, 32, sub, 0)

            liness = (lines, lines2)
            rowss = (rows_v, rows_v2)
            fill_lines(0, lines)
            pltpu.async_copy(table_hbm.at[lines], rows_v, sem)
            fill_lines(1, lines2)
            pltpu.async_copy(table_hbm.at[lines2], rows_v2, sem)

            def pair(t, _):
                for b in range(2):
                    r = 2 * t + b
                    pltpu.make_async_copy(
                        table_hbm.at[liness[b]], rowss[b], sem
                    ).wait()
                    transpose_row(r, rowss[b])

                    @pl.when(t < 3)
                    def _():
                        fill_lines(r + 2, liness[b])
                        pltpu.async_copy(
                            table_hbm.at[liness[b]], rowss[b], sem
                        )

                    pltpu.sync_copy(
                        xu,
                        out_hbm.at[
                            so * 8 + r,
                            :,
                            pl.ds(pl.multiple_of(j * 128, 128), 128),
                        ],
                    )
                return 0

            lax.fori_loop(0, 4, pair, 0)

        return 0

    lax.fori_loop(0, 7, unit, 0)


@functools.cache
def _sc_gatherx():
    return pl.kernel(
        _sc_gatherx_body,
        out_type=jax.ShapeDtypeStruct((_SEQ, _EMBED_DIM, _BATCH), jnp.float32),
        mesh=plsc.VectorSubcoreMesh(core_axis_name="c", subcore_axis_name="s"),
        scratch_types=[
            pltpu.VMEM((8, 128), jnp.int32),
            pltpu.VMEM((128,), jnp.int32),
            pltpu.VMEM((128,), jnp.int32),
            pltpu.VMEM((128, 128), jnp.float32),
            pltpu.VMEM((128, 128), jnp.float32),
            pltpu.VMEM((_EMBED_DIM, 128), jnp.float32),
            pltpu.SemaphoreType.DMA,
        ],
        compiler_params=pltpu.CompilerParams(
            use_tc_tiling_on_sc=True, needs_layout_passes=False
        ),
    )


_QB = 2


def _tc_mask_body(mask_ref, idx_ref, attn_ref, loss_ref):
    keep = idx_ref[...] != 0
    attn_ref[...] = mask_ref[...] * keep[None].astype(jnp.float32)
    loss_ref[...] = keep


def _tc_mask(mask_t, idx_t):
    return pl.pallas_call(
        _tc_mask_body,
        grid=(_SEQ // _QB,),
        in_specs=[
            pl.BlockSpec((_QB, _SEQ, _BATCH), lambda i: (i, 0, 0)),
            pl.BlockSpec((_SEQ, _BATCH), lambda i: (0, 0)),
        ],
        out_specs=[
            pl.BlockSpec((_QB, _SEQ, _BATCH), lambda i: (i, 0, 0)),
            pl.BlockSpec((_SEQ, _BATCH), lambda i: (0, 0)),
        ],
        out_shape=[
            jax.ShapeDtypeStruct((_SEQ, _SEQ, _BATCH), jnp.float32),
            jax.ShapeDtypeStruct((_SEQ, _BATCH), jnp.bool_),
        ],
        cost_estimate=pl.CostEstimate(
            flops=2 * _SEQ * _SEQ * _BATCH,
            bytes_accessed=2 * _SEQ * _SEQ * _BATCH * 4,
            transcendentals=0,
        ),
    )(mask_t, idx_t)


def kernel(inputs, mask, table):
    mask_t = jnp.transpose(mask.reshape(_BATCH, _SEQ, _SEQ), (1, 2, 0))
    attn_t, loss_t = _tc_mask(mask_t, inputs.T)
    tl2 = _sc_table()(table.T)
    xt3 = _sc_gatherx()(inputs.T, tl2)
    attn = jnp.transpose(attn_t, (2, 0, 1)).reshape(_BATCH, 1, _SEQ, _SEQ)
    return (
        jnp.transpose(xt3, (2, 0, 1)),
        attn,
        loss_t.T,
    )
